# baseline (speedup 1.0000x reference)
import functools

import jax
import jax.numpy as jnp
from jax.experimental import pallas as pl
from jax.experimental.pallas import tpu as pltpu


def _frame_kernel(x_ref, msk_ref, mskrow_ref, w0, b0, w1, b1, w2, b2, wl, bl,
                  o_ref, xc0, xc1, xc2, *, H, W, D):
    Hp, Wp, Dp = H + 2, W + 2, D + 2
    Sp = Hp * Wp * Dp
    WpDp = Wp * Dp
    M = WpDp + Dp + 1
    SpM = Sp + 2 * M
    S2 = Sp - 2 * M
    n_real = H * W * D

    msk = msk_ref[...]

    def conv9(xc_ref, w_ref, b_ref):
        cout = w_ref.shape[2]
        offs = [2 * M + (kh - 1) * WpDp + (kw - 1) * Dp
                for kh in range(3) for kw in range(3)]
        acc0 = jnp.zeros((S2, cout), jnp.float32)
        acc1 = jnp.zeros((S2, cout), jnp.float32)
        for j, off in enumerate(offs):
            d = jnp.dot(xc_ref[pl.ds(off, S2), :], w_ref[j],
                        preferred_element_type=jnp.float32)
            if j % 2 == 0:
                acc0 = acc0 + d
            else:
                acc1 = acc1 + d
        return jnp.maximum(acc0 + acc1 + b_ref[...], 0.0)

    def store_bands(xc_ref, am, c):
        zt = jnp.zeros((M + 2, 3 * c), xc_ref.dtype)
        xc_ref[pl.ds(M, M + 2), :] = zt
        xc_ref[pl.ds(2 * M + S2 - 2, M + 2), :] = zt
        xc_ref[pl.ds(2 * M + 1, S2), 0:c] = am
        xc_ref[pl.ds(2 * M, S2), c:2 * c] = am
        xc_ref[pl.ds(2 * M - 1, S2), 2 * c:3 * c] = am

    xv = x_ref[0]
    zr = jnp.zeros((1, 9), xv.dtype)
    xc0[pl.ds(0, 1), :] = zr
    xc0[pl.ds(SpM - 1, 1), :] = zr
    xc0[pl.ds(1, SpM - 1), 0:3] = xv[0:SpM - 1]
    xc0[:, 3:6] = xv
    xc0[pl.ds(0, SpM - 1), 6:9] = xv[1:SpM]
    a = conv9(xc0, w0, b0)

    store_bands(xc1, (a * msk).astype(xc1.dtype), 32)
    a = conv9(xc1, w1, b1)

    store_bands(xc2, (a * msk).astype(xc2.dtype), 64)
    a = conv9(xc2, w2, b2)

    pooled = jnp.dot(mskrow_ref[...], a,
                     preferred_element_type=jnp.float32) * jnp.float32(1.0 / n_real)
    feat = jnp.dot(pooled, wl[...], preferred_element_type=jnp.float32) + bl[...]
    o_ref[0] = feat


def kernel(x, conv_w0, conv_w1, conv_w2, conv_b0, conv_b1, conv_b2, proj_w, proj_b):
    B, H, W, D, C, T = x.shape
    Hp, Wp, Dp = H + 2, W + 2, D + 2
    Sp = Hp * Wp * Dp
    M = Wp * Dp + Dp + 1
    SpM = Sp + 2 * M
    N = B * T
    E = proj_w.shape[-1]

    xf = jnp.transpose(x, (0, 5, 1, 2, 3, 4)).reshape(N, H, W, D, C)
    xf = jnp.pad(xf, ((0, 0), (1, 1), (1, 1), (1, 1), (0, 0)))
    xf = xf.reshape(N, Sp, C)
    xf = jnp.pad(xf, ((0, 0), (M, M), (0, 0))).astype(jnp.bfloat16)

    interior = (
        jnp.zeros((Hp, Wp, Dp), jnp.float32)
        .at[1:H + 1, 1:W + 1, 1:D + 1].set(1.0)
        .reshape(Sp, 1)
    )
    S2 = Sp - 2 * M
    interior = interior[M:M + S2]
    interior_row = interior.reshape(1, S2)

    w0 = conv_w0.reshape(9, 3 * 3, 32).astype(jnp.bfloat16)
    w1 = conv_w1.reshape(9, 3 * 32, 64).astype(jnp.bfloat16)
    w2 = conv_w2.reshape(9, 3 * 64, 128).astype(jnp.bfloat16)

    body = functools.partial(_frame_kernel, H=H, W=W, D=D)

    in_specs = [
        pl.BlockSpec((1, SpM, C), lambda i: (i, 0, 0)),
        pl.BlockSpec((S2, 1), lambda i: (0, 0)),
        pl.BlockSpec((1, S2), lambda i: (0, 0)),
        pl.BlockSpec(w0.shape, lambda i: (0, 0, 0)),
        pl.BlockSpec(conv_b0.shape, lambda i: (0, 0)),
        pl.BlockSpec(w1.shape, lambda i: (0, 0, 0)),
        pl.BlockSpec(conv_b1.shape, lambda i: (0, 0)),
        pl.BlockSpec(w2.shape, lambda i: (0, 0, 0)),
        pl.BlockSpec(conv_b2.shape, lambda i: (0, 0)),
        pl.BlockSpec(proj_w.shape, lambda i: (0, 0)),
        pl.BlockSpec(proj_b.shape, lambda i: (0, 0)),
    ]

    out = pl.pallas_call(
        body,
        out_shape=jax.ShapeDtypeStruct((N, 1, E), jnp.float32),
        grid=(N,),
        in_specs=in_specs,
        out_specs=pl.BlockSpec((1, 1, E), lambda i: (i, 0, 0)),
        scratch_shapes=[
            pltpu.VMEM((SpM, 9), jnp.bfloat16),
            pltpu.VMEM((SpM, 3 * 32), jnp.bfloat16),
            pltpu.VMEM((SpM, 3 * 64), jnp.bfloat16),
        ],
        compiler_params=pltpu.CompilerParams(dimension_semantics=("parallel",)),
    )(xf, interior, interior_row, w0, conv_b0, w1, conv_b1, w2, conv_b2,
      proj_w, proj_b)

    out = out.reshape(B, T, E)
    return jnp.transpose(out, (0, 2, 1))
